# baseline (device time: 89479 ns/iter reference)
import jax
import jax.numpy as jnp
from jax import lax
from jax.experimental import pallas as pl
from jax.experimental.pallas import tpu as pltpu

N_DEV = 4


def kernel(x, w_mat, scale_x, scale_w):
    m_total, k_per = x.shape
    _, n = w_mat.shape
    m_per = m_total // N_DEV
    nh = n // 2

    def body(x_ref, w_ref, sx_ref, sw_ref, out_ref,
             a1s, a1r, b1s, b1r, a2s, b2s,
             send_sems, recv_sems):
        p = lax.axis_index("i")
        q = jnp.bitwise_xor(p, 1)
        xp = 3 - p

        barrier_sem = pltpu.get_barrier_semaphore()
        for nbr in (q, xp):
            pl.semaphore_signal(
                barrier_sem, inc=1,
                device_id=(nbr,), device_id_type=pl.DeviceIdType.MESH,
            )
        pl.semaphore_wait(barrier_sem, 2)

        def partial(c, lo, hi):
            xs = x_ref[pl.ds(c * m_per, m_per), :]
            return lax.dot_general(
                xs, w_ref[:, lo:hi],
                dimension_numbers=(((1,), (0,)), ((), ())),
                preferred_element_type=jnp.int32,
            )

        bf16 = jnp.bfloat16

        def rdma(src, dst, sem_idx, target):
            return pltpu.make_async_remote_copy(
                src_ref=src, dst_ref=dst,
                send_sem=send_sems.at[sem_idx], recv_sem=recv_sems.at[sem_idx],
                device_id=(target,), device_id_type=pl.DeviceIdType.MESH,
            )

        a1_0 = rdma(a1s.at[0], a1r.at[0], 0, xp)
        a1_1 = rdma(a1s.at[1], a1r.at[1], 1, xp)
        b1_0 = rdma(b1s.at[0], b1r.at[0], 2, q)
        b1_1 = rdma(b1s.at[1], b1r.at[1], 3, q)
        a2 = rdma(a2s, a1r.at[0], 4, q)
        b2 = rdma(b2s, b1r.at[0], 5, xp)

        a1s[0, :, :] = partial(3 - q, 0, nh).astype(bf16)
        a1_0.start()
        b1s[0, :, :] = partial(3 - q, nh, n).astype(bf16)
        b1_0.start()
        a1s[1, :, :] = partial(3 - p, 0, nh).astype(bf16)
        a1_1.start()
        b1s[1, :, :] = partial(q, nh, n).astype(bf16)
        b1_1.start()

        a2s[:, :] = partial(q, 0, nh).astype(bf16)
        b2s[:, :] = partial(3 - p, nh, n).astype(bf16)

        a1_0.wait_recv()
        a2s[:, :] = (
            a2s[:, :].astype(jnp.float32) + a1r[0, :, :].astype(jnp.float32)
        ).astype(bf16)
        a2.start()
        b1_0.wait_recv()
        b2s[:, :] = (
            b2s[:, :].astype(jnp.float32) + b1r[0, :, :].astype(jnp.float32)
        ).astype(bf16)
        b2.start()

        s = sx_ref[0, 0] * sw_ref[0, 0]

        pa = partial(p, 0, nh).astype(jnp.float32)
        a1_1.wait_recv()
        a2.wait_recv()
        out_ref[:, 0:nh] = (
            pa + a1r[1, :, :].astype(jnp.float32)
            + a1r[0, :, :].astype(jnp.float32)
        ) * s

        pb = partial(p, nh, n).astype(jnp.float32)
        b1_1.wait_recv()
        b2.wait_recv()
        out_ref[:, nh:n] = (
            pb + b1r[1, :, :].astype(jnp.float32)
            + b1r[0, :, :].astype(jnp.float32)
        ) * s

        for d in (a1_0, a1_1, b1_0, b1_1, a2, b2):
            d.wait_send()

    return pl.pallas_call(
        body,
        out_shape=jax.ShapeDtypeStruct((m_per, n), jnp.float32),
        in_specs=[pl.BlockSpec(memory_space=pltpu.VMEM)] * 4,
        out_specs=pl.BlockSpec(memory_space=pltpu.VMEM),
        scratch_shapes=[
            pltpu.VMEM((2, m_per, nh), jnp.bfloat16),
            pltpu.VMEM((2, m_per, nh), jnp.bfloat16),
            pltpu.VMEM((2, m_per, nh), jnp.bfloat16),
            pltpu.VMEM((2, m_per, nh), jnp.bfloat16),
            pltpu.VMEM((m_per, nh), jnp.bfloat16),
            pltpu.VMEM((m_per, nh), jnp.bfloat16),
            pltpu.SemaphoreType.DMA((6,)),
            pltpu.SemaphoreType.DMA((6,)),
        ],
        compiler_params=pltpu.CompilerParams(collective_id=0),
    )(x, w_mat, scale_x.reshape(1, 1), scale_w.reshape(1, 1))


# device time: 74392 ns/iter; 1.2028x vs baseline; 1.2028x over previous
import jax
import jax.numpy as jnp
from jax import lax
from jax.experimental import pallas as pl
from jax.experimental.pallas import tpu as pltpu

N_DEV = 4


def kernel(x, w_mat, scale_x, scale_w):
    m_total, k_per = x.shape
    _, n = w_mat.shape
    m_per = m_total // N_DEV
    kh = k_per // 2

    def body(x_ref, w_ref, sx_ref, sw_ref, out_ref,
             xr, wrA, wrB, send_sems, recv_sems):
        p = lax.axis_index("i")
        q = jnp.bitwise_xor(p, 1)
        xp = 3 - p
        dg = jnp.bitwise_xor(p, 2)

        barrier_sem = pltpu.get_barrier_semaphore()
        for nbr in (q, xp):
            pl.semaphore_signal(
                barrier_sem, inc=1,
                device_id=(nbr,), device_id_type=pl.DeviceIdType.MESH,
            )
        pl.semaphore_wait(barrier_sem, 2)

        def rdma(src, dst, sem_idx, target):
            return pltpu.make_async_remote_copy(
                src_ref=src, dst_ref=dst,
                send_sem=send_sems.at[sem_idx], recv_sem=recv_sems.at[sem_idx],
                device_id=(target,), device_id_type=pl.DeviceIdType.MESH,
            )

        wA_src = w_ref.at[pl.ds(0, kh), :]
        wB_src = w_ref.at[pl.ds(kh, kh), :]

        a1 = rdma(wA_src, wrA.at[0], 0, xp)
        b1 = rdma(wB_src, wrB.at[0], 1, q)
        x_q = rdma(x_ref.at[pl.ds(q * m_per, m_per), :], xr.at[0], 2, q)
        x_xp = rdma(x_ref.at[pl.ds(xp * m_per, m_per), :], xr.at[1], 3, xp)
        x_dg = rdma(x_ref.at[pl.ds(dg * m_per, m_per), :], xr.at[2], 4, dg)
        a2a = rdma(wA_src, wrA.at[1], 5, q)
        a2b = rdma(wrA.at[0], wrA.at[2], 6, q)
        b2a = rdma(wB_src, wrB.at[1], 7, xp)
        b2b = rdma(wrB.at[0], wrB.at[2], 8, xp)

        a1.start()
        b1.start()
        x_q.start()
        x_xp.start()
        x_dg.start()

        def dot(xs, ws):
            return lax.dot_general(
                xs, ws,
                dimension_numbers=(((1,), (0,)), ((), ())),
                preferred_element_type=jnp.int32,
            )

        out_ref[:, :] = dot(
            x_ref[pl.ds(p * m_per, m_per), :], w_ref[:, :]
        ).astype(jnp.float32)

        a1.wait_recv()
        a2a.start()
        a2b.start()
        b1.wait_recv()
        b2a.start()
        b2b.start()

        x_q.wait_recv()
        d_qb = dot(xr[0, :, kh:k_per], wrB[0, :, :])
        x_xp.wait_recv()
        d_xpa = dot(xr[1, :, 0:kh], wrA[0, :, :])
        out_ref[:, :] = out_ref[:, :] + (d_qb + d_xpa).astype(jnp.float32)

        a2a.wait_recv()
        d_qa = dot(xr[0, :, 0:kh], wrA[1, :, :])
        b2a.wait_recv()
        d_xpb = dot(xr[1, :, kh:k_per], wrB[1, :, :])
        out_ref[:, :] = out_ref[:, :] + (d_qa + d_xpb).astype(jnp.float32)

        x_dg.wait_recv()
        a2b.wait_recv()
        b2b.wait_recv()
        d_dg = dot(xr[2, :, 0:kh], wrA[2, :, :]) + dot(
            xr[2, :, kh:k_per], wrB[2, :, :]
        )
        s = sx_ref[0, 0] * sw_ref[0, 0]
        out_ref[:, :] = (out_ref[:, :] + d_dg.astype(jnp.float32)) * s

        for d in (a1, b1, x_q, x_xp, x_dg, a2a, a2b, b2a, b2b):
            d.wait_send()
        for nbr in (q, xp):
            pl.semaphore_signal(
                barrier_sem, inc=1,
                device_id=(nbr,), device_id_type=pl.DeviceIdType.MESH,
            )
        pl.semaphore_wait(barrier_sem, 2)

    return pl.pallas_call(
        body,
        out_shape=jax.ShapeDtypeStruct((m_per, n), jnp.float32),
        in_specs=[pl.BlockSpec(memory_space=pltpu.VMEM)] * 4,
        out_specs=pl.BlockSpec(memory_space=pltpu.VMEM),
        scratch_shapes=[
            pltpu.VMEM((3, m_per, k_per), jnp.int8),
            pltpu.VMEM((3, kh, n), jnp.int8),
            pltpu.VMEM((3, kh, n), jnp.int8),
            pltpu.SemaphoreType.DMA((9,)),
            pltpu.SemaphoreType.DMA((9,)),
        ],
        compiler_params=pltpu.CompilerParams(collective_id=0),
    )(x, w_mat, scale_x.reshape(1, 1), scale_w.reshape(1, 1))


# device time: 71387 ns/iter; 1.2534x vs baseline; 1.0421x over previous
import jax
import jax.numpy as jnp
from jax import lax
from jax.experimental import pallas as pl
from jax.experimental.pallas import tpu as pltpu

N_DEV = 4


def kernel(x, w_mat, scale_x, scale_w):
    m_total, k_per = x.shape
    _, n = w_mat.shape
    m_per = m_total // N_DEV
    kh = k_per // 2

    def body(x_ref, w_ref, sx_ref, sw_ref, out_ref,
             xr, wrA, wrB, send_sems, recv_sems):
        p = lax.axis_index("i")
        q = jnp.bitwise_xor(p, 1)
        xp = 3 - p
        dg = jnp.bitwise_xor(p, 2)

        barrier_sem = pltpu.get_barrier_semaphore()
        for nbr in (q, xp):
            pl.semaphore_signal(
                barrier_sem, inc=1,
                device_id=(nbr,), device_id_type=pl.DeviceIdType.MESH,
            )
        pl.semaphore_wait(barrier_sem, 2)

        def rdma(src, dst, sem_idx, target):
            return pltpu.make_async_remote_copy(
                src_ref=src, dst_ref=dst,
                send_sem=send_sems.at[sem_idx], recv_sem=recv_sems.at[sem_idx],
                device_id=(target,), device_id_type=pl.DeviceIdType.MESH,
            )

        wA_src = w_ref.at[pl.ds(0, kh), :]
        wB_src = w_ref.at[pl.ds(kh, kh), :]

        a1 = rdma(wA_src, wrA.at[0], 0, xp)
        b1 = rdma(wB_src, wrB.at[0], 1, q)
        x_q = rdma(x_ref.at[pl.ds(q * m_per, m_per), :], xr.at[0], 2, q)
        x_xp = rdma(x_ref.at[pl.ds(xp * m_per, m_per), :], xr.at[1], 3, xp)
        x_dg = rdma(x_ref.at[pl.ds(dg * m_per, m_per), :], xr.at[2], 4, dg)
        nh2 = n // 2
        a2a = rdma(wA_src, wrA.at[1], 5, q)
        b2a = rdma(wB_src, wrB.at[1], 7, xp)
        a2b0 = rdma(wrA.at[0, :, pl.ds(0, nh2)], wrA.at[2, :, pl.ds(0, nh2)],
                    6, q)
        a2b1 = rdma(wrA.at[0, :, pl.ds(nh2, nh2)],
                    wrA.at[2, :, pl.ds(nh2, nh2)], 9, q)
        b2b0 = rdma(wrB.at[0, :, pl.ds(0, nh2)], wrB.at[2, :, pl.ds(0, nh2)],
                    8, xp)
        b2b1 = rdma(wrB.at[0, :, pl.ds(nh2, nh2)],
                    wrB.at[2, :, pl.ds(nh2, nh2)], 10, xp)

        a1.start()
        b1.start()
        x_q.start()
        x_xp.start()
        x_dg.start()

        def dot(xs, ws):
            return lax.dot_general(
                xs, ws,
                dimension_numbers=(((1,), (0,)), ((), ())),
                preferred_element_type=jnp.int32,
            )

        out_ref[:, :] = dot(
            x_ref[pl.ds(p * m_per, m_per), :], w_ref[:, :]
        ).astype(jnp.float32)

        a1.wait_recv()
        a2a.start()
        a2b0.start()
        a2b1.start()
        b1.wait_recv()
        b2a.start()
        b2b0.start()
        b2b1.start()

        x_q.wait_recv()
        d_qb = dot(xr[0, :, kh:k_per], wrB[0, :, :])
        x_xp.wait_recv()
        d_xpa = dot(xr[1, :, 0:kh], wrA[0, :, :])
        out_ref[:, :] = out_ref[:, :] + (d_qb + d_xpa).astype(jnp.float32)

        a2a.wait_recv()
        d_qa = dot(xr[0, :, 0:kh], wrA[1, :, :])
        b2a.wait_recv()
        d_xpb = dot(xr[1, :, kh:k_per], wrB[1, :, :])
        out_ref[:, :] = out_ref[:, :] + (d_qa + d_xpb).astype(jnp.float32)

        s = sx_ref[0, 0] * sw_ref[0, 0]
        x_dg.wait_recv()
        a2b0.wait_recv()
        d_dg0a = dot(xr[2, :, 0:kh], wrA[2, :, 0:nh2])
        b2b0.wait_recv()
        d_dg0b = dot(xr[2, :, kh:k_per], wrB[2, :, 0:nh2])
        out_ref[:, 0:nh2] = (
            out_ref[:, 0:nh2] + (d_dg0a + d_dg0b).astype(jnp.float32)
        ) * s
        a2b1.wait_recv()
        d_dg1a = dot(xr[2, :, 0:kh], wrA[2, :, nh2:n])
        b2b1.wait_recv()
        d_dg1b = dot(xr[2, :, kh:k_per], wrB[2, :, nh2:n])
        out_ref[:, nh2:n] = (
            out_ref[:, nh2:n] + (d_dg1a + d_dg1b).astype(jnp.float32)
        ) * s

        for d in (a1, b1, x_q, x_xp, x_dg, a2a, a2b0, a2b1, b2a, b2b0, b2b1):
            d.wait_send()
        for nbr in (q, xp):
            pl.semaphore_signal(
                barrier_sem, inc=1,
                device_id=(nbr,), device_id_type=pl.DeviceIdType.MESH,
            )
        pl.semaphore_wait(barrier_sem, 2)

    return pl.pallas_call(
        body,
        out_shape=jax.ShapeDtypeStruct((m_per, n), jnp.float32),
        in_specs=[pl.BlockSpec(memory_space=pltpu.VMEM)] * 4,
        out_specs=pl.BlockSpec(memory_space=pltpu.VMEM),
        scratch_shapes=[
            pltpu.VMEM((3, m_per, k_per), jnp.int8),
            pltpu.VMEM((3, kh, n), jnp.int8),
            pltpu.VMEM((3, kh, n), jnp.int8),
            pltpu.SemaphoreType.DMA((11,)),
            pltpu.SemaphoreType.DMA((11,)),
        ],
        compiler_params=pltpu.CompilerParams(collective_id=0),
    )(x, w_mat, scale_x.reshape(1, 1), scale_w.reshape(1, 1))
